# baseline (speedup 1.0000x reference)
import functools

import jax
import jax.numpy as jnp
from jax import lax
from jax.experimental import pallas as pl
from jax.experimental.pallas import tpu as pltpu
from jax.experimental.pallas import tpu_sc as plsc

N = 10000
E = 320000
D = 128
NC = 2
NS = 16
NW = NC * NS
EPT = E // NW
K = 80
NITER = EPT // K
GRP = 3
NGRP = NITER // GRP
TAIL = NITER - NGRP * GRP
RPT = N // NS
BLK = 2000
GRID = N // BLK

@functools.cache
def _sc_kernels():
    mesh = plsc.VectorSubcoreMesh(
        core_axis_name="c", subcore_axis_name="s",
        num_cores=NC, num_subcores=NS,
    )

    @functools.partial(
        pl.kernel,
        out_type=jax.ShapeDtypeStruct((NW, N), jnp.float32),
        mesh=mesh,
        compiler_params=pltpu.CompilerParams(needs_layout_passes=False),
        scratch_types=[
            pltpu.VMEM((EPT // 16, 16), jnp.int32),
            pltpu.VMEM((N,), jnp.float32),
        ],
    )
    def _deg_kernel(dst_hbm, out_hbm, idx_v, deg_v):
        cid = lax.axis_index("c")
        sid = lax.axis_index("s")
        wid = sid * NC + cid
        pltpu.sync_copy(dst_hbm.at[wid], idx_v)

        def zero_body(i, c):
            deg_v[pl.ds(i * 16, 16)] = jnp.zeros((16,), jnp.float32)
            return c

        lax.fori_loop(0, N // 16, zero_body, 0, unroll=4)

        ones = jnp.ones((16,), jnp.float32)

        def body(i, c):
            plsc.addupdate_scatter(deg_v, [idx_v[i]], ones)
            return c

        lax.fori_loop(0, EPT // 16, body, 0, unroll=4)
        pltpu.sync_copy(deg_v, out_hbm.at[wid])

    @functools.partial(
        pl.kernel,
        out_type=jax.ShapeDtypeStruct((NC, N, D), jnp.float32),
        mesh=mesh,
        compiler_params=pltpu.CompilerParams(
            needs_layout_passes=False, use_tc_tiling_on_sc=False),
        scratch_types=[
            pltpu.VMEM((NITER, K), jnp.int32),
            pltpu.VMEM((NITER, K), jnp.int32),
            pltpu.VMEM((GRP, K, D), jnp.float32),
            pltpu.VMEM_SHARED((N, D), jnp.float32),
            [pltpu.SemaphoreType.DMA] * GRP,
            [pltpu.SemaphoreType.DMA] * GRP,
            pltpu.SemaphoreType.DMA,
        ],
    )
    def _agg_kernel(y_hbm, z_hbm, src_hbm, dst_hbm, out_hbm,
                    si_v, di_v, rows_v, acc_sh, gsems, ssems, seed_sem):
        cid = lax.axis_index("c")
        sid = lax.axis_index("s")
        wid = sid * NC + cid
        base = sid * RPT

        @pl.when(cid == 0)
        def _():
            pltpu.async_copy(y_hbm.at[pl.ds(base, RPT)],
                             acc_sh.at[pl.ds(base, RPT)], seed_sem)

        @pl.when(cid != 0)
        def _():
            pltpu.async_copy(z_hbm, acc_sh.at[pl.ds(base, RPT)], seed_sem)

        pltpu.sync_copy(src_hbm.at[wid], si_v)
        pltpu.sync_copy(dst_hbm.at[wid], di_v)
        pltpu.make_async_copy(
            z_hbm, acc_sh.at[pl.ds(base, RPT)], seed_sem).wait()
        plsc.subcore_barrier()

        def group(j0, n):
            gh = [
                pltpu.async_copy(
                    y_hbm.at[si_v.at[j0 + b]], rows_v.at[b], gsems[b])
                for b in range(n)
            ]
            sh = []
            for b in range(n):
                gh[b].wait()
                sh.append(pltpu.async_copy(
                    rows_v.at[b], acc_sh.at[di_v.at[j0 + b]], ssems[b],
                    add=True))
            for h in sh:
                h.wait()

        def body(i, c):
            group(i * GRP, GRP)
            return c

        lax.fori_loop(0, NGRP, body, 0)
        if TAIL:
            group(NGRP * GRP, TAIL)
        plsc.subcore_barrier()
        pltpu.sync_copy(acc_sh.at[pl.ds(base, RPT)],
                        out_hbm.at[cid, pl.ds(base, RPT)])

    return _deg_kernel, _agg_kernel


def _dinv_of(degp_blk):
    return lax.rsqrt(1.0 + jnp.sum(degp_blk, axis=1, keepdims=True))


def _stage_in_body(degp_ref, x_ref, w_ref, y_ref):
    dinv = _dinv_of(degp_ref[...])
    xw = jnp.dot(x_ref[...], w_ref[...], preferred_element_type=jnp.float32)
    y_ref[...] = xw * dinv


_stage_in = pl.pallas_call(
    _stage_in_body,
    grid=(GRID,),
    in_specs=[
        pl.BlockSpec((BLK, NW), lambda i: (i, 0)),
        pl.BlockSpec((BLK, D), lambda i: (i, 0)),
        pl.BlockSpec((D, D), lambda i: (0, 0)),
    ],
    out_specs=pl.BlockSpec((BLK, D), lambda i: (i, 0)),
    out_shape=jax.ShapeDtypeStruct((N, D), jnp.float32),
)


def _stage_mid_body(degp_ref, agg_ref, b_ref, w_ref, y_ref):
    dinv = _dinv_of(degp_ref[...])
    s = agg_ref[0] + agg_ref[1]
    h = jnp.maximum(s * dinv + b_ref[...], 0.0)
    y_ref[...] = (
        jnp.dot(h, w_ref[...], preferred_element_type=jnp.float32) * dinv
    )


_stage_mid = pl.pallas_call(
    _stage_mid_body,
    grid=(GRID,),
    in_specs=[
        pl.BlockSpec((BLK, NW), lambda i: (i, 0)),
        pl.BlockSpec((NC, BLK, D), lambda i: (0, i, 0)),
        pl.BlockSpec((1, D), lambda i: (0, 0)),
        pl.BlockSpec((D, D), lambda i: (0, 0)),
    ],
    out_specs=pl.BlockSpec((BLK, D), lambda i: (i, 0)),
    out_shape=jax.ShapeDtypeStruct((N, D), jnp.float32),
)


def _make_stage_out(dout):
    def _stage_out_body(degp_ref, agg_ref, b_ref, wl_ref, bl_ref,
                        nemb_ref, gsum_ref, gemb_ref):
        i = pl.program_id(0)
        dinv = _dinv_of(degp_ref[...])
        s = agg_ref[0] + agg_ref[1]
        nemb = s * dinv + b_ref[...]
        nemb_ref[...] = nemb

        @pl.when(i == 0)
        def _():
            gsum_ref[...] = jnp.zeros_like(gsum_ref)

        gsum_ref[...] += jnp.sum(nemb, axis=0, keepdims=True)

        @pl.when(i == GRID - 1)
        def _():
            g = gsum_ref[...] * (1.0 / N)
            gemb_ref[...] = (
                lax.dot_general(g, wl_ref[...], (((1,), (1,)), ((), ())),
                                preferred_element_type=jnp.float32)
                + bl_ref[...]
            )

    return pl.pallas_call(
        _stage_out_body,
        grid=(GRID,),
        in_specs=[
            pl.BlockSpec((BLK, NW), lambda i: (i, 0)),
            pl.BlockSpec((NC, BLK, D), lambda i: (0, i, 0)),
            pl.BlockSpec((1, D), lambda i: (0, 0)),
            pl.BlockSpec((dout, D), lambda i: (0, 0)),
            pl.BlockSpec((1, dout), lambda i: (0, 0)),
        ],
        out_specs=[
            pl.BlockSpec((BLK, D), lambda i: (i, 0)),
            pl.BlockSpec((1, D), lambda i: (0, 0)),
            pl.BlockSpec((1, dout), lambda i: (0, 0)),
        ],
        out_shape=[
            jax.ShapeDtypeStruct((N, D), jnp.float32),
            jax.ShapeDtypeStruct((1, D), jnp.float32),
            jax.ShapeDtypeStruct((1, dout), jnp.float32),
        ],
    )


def kernel(x, edge_index, batch, W1, b1, W2, b2, W3, b3, Wl, bl):
    dout = Wl.shape[0]
    src = edge_index[0].reshape(NW, NITER, K)
    dst = edge_index[1].reshape(NW, NITER, K)
    dst16 = edge_index[1].reshape(NW, EPT // 16, 16)
    zrows = jnp.zeros((RPT, D), jnp.float32)
    deg_kernel, agg_kernel = _sc_kernels()

    degp = jnp.transpose(deg_kernel(dst16))

    y1 = _stage_in(degp, x, W1)
    p1 = agg_kernel(y1, zrows, src, dst)
    y2 = _stage_mid(degp, p1, b1.reshape(1, D), W2)
    p2 = agg_kernel(y2, zrows, src, dst)
    y3 = _stage_mid(degp, p2, b2.reshape(1, D), W3)
    p3 = agg_kernel(y3, zrows, src, dst)
    nemb, _gsum, gemb = _make_stage_out(dout)(
        degp, p3, b3.reshape(1, D), Wl, bl.reshape(1, dout)
    )
    return (gemb, nemb)


# speedup vs baseline: 1.0418x; 1.0418x over previous
import functools

import jax
import jax.numpy as jnp
from jax import lax
from jax.experimental import pallas as pl
from jax.experimental.pallas import tpu as pltpu
from jax.experimental.pallas import tpu_sc as plsc

N = 10000
E = 320000
D = 128
NC = 2
NS = 16
NW = NC * NS
EPT = E // NW
K = 40
NITER = EPT // K
GRP = 6
NGRP = NITER // GRP
TAIL = NITER - NGRP * GRP
RPT = N // NS
BLK = 2000
GRID = N // BLK

@functools.cache
def _sc_kernels():
    mesh = plsc.VectorSubcoreMesh(
        core_axis_name="c", subcore_axis_name="s",
        num_cores=NC, num_subcores=NS,
    )

    @functools.partial(
        pl.kernel,
        out_type=jax.ShapeDtypeStruct((NW, N), jnp.float32),
        mesh=mesh,
        compiler_params=pltpu.CompilerParams(needs_layout_passes=False),
        scratch_types=[
            pltpu.VMEM((EPT // 16, 16), jnp.int32),
            pltpu.VMEM((N,), jnp.float32),
        ],
    )
    def _deg_kernel(dst_hbm, out_hbm, idx_v, deg_v):
        cid = lax.axis_index("c")
        sid = lax.axis_index("s")
        wid = sid * NC + cid
        pltpu.sync_copy(dst_hbm.at[wid], idx_v)

        def zero_body(i, c):
            deg_v[pl.ds(i * 16, 16)] = jnp.zeros((16,), jnp.float32)
            return c

        lax.fori_loop(0, N // 16, zero_body, 0, unroll=4)

        ones = jnp.ones((16,), jnp.float32)

        def body(i, c):
            plsc.addupdate_scatter(deg_v, [idx_v[i]], ones)
            return c

        lax.fori_loop(0, EPT // 16, body, 0, unroll=4)
        pltpu.sync_copy(deg_v, out_hbm.at[wid])

    @functools.partial(
        pl.kernel,
        out_type=jax.ShapeDtypeStruct((NC, N, D), jnp.float32),
        mesh=mesh,
        compiler_params=pltpu.CompilerParams(
            needs_layout_passes=False, use_tc_tiling_on_sc=False),
        scratch_types=[
            pltpu.VMEM((NITER, K), jnp.int32),
            pltpu.VMEM((NITER, K), jnp.int32),
            pltpu.VMEM((GRP, K, D), jnp.float32),
            pltpu.VMEM_SHARED((N, D), jnp.float32),
            [pltpu.SemaphoreType.DMA] * GRP,
            [pltpu.SemaphoreType.DMA] * GRP,
            pltpu.SemaphoreType.DMA,
        ],
    )
    def _agg_kernel(y_hbm, z_hbm, src_hbm, dst_hbm, out_hbm,
                    si_v, di_v, rows_v, acc_sh, gsems, ssems, seed_sem):
        cid = lax.axis_index("c")
        sid = lax.axis_index("s")
        wid = sid * NC + cid
        base = sid * RPT

        @pl.when(cid == 0)
        def _():
            pltpu.async_copy(y_hbm.at[pl.ds(base, RPT)],
                             acc_sh.at[pl.ds(base, RPT)], seed_sem)

        @pl.when(cid != 0)
        def _():
            pltpu.async_copy(z_hbm, acc_sh.at[pl.ds(base, RPT)], seed_sem)

        pltpu.sync_copy(src_hbm.at[wid], si_v)
        pltpu.sync_copy(dst_hbm.at[wid], di_v)
        pltpu.make_async_copy(
            z_hbm, acc_sh.at[pl.ds(base, RPT)], seed_sem).wait()
        plsc.subcore_barrier()

        def group(j0, n):
            gh = [
                pltpu.async_copy(
                    y_hbm.at[si_v.at[j0 + b]], rows_v.at[b], gsems[b])
                for b in range(n)
            ]
            sh = []
            for b in range(n):
                gh[b].wait()
                sh.append(pltpu.async_copy(
                    rows_v.at[b], acc_sh.at[di_v.at[j0 + b]], ssems[b],
                    add=True))
            for h in sh:
                h.wait()

        def body(i, c):
            group(i * GRP, GRP)
            return c

        lax.fori_loop(0, NGRP, body, 0)
        if TAIL:
            group(NGRP * GRP, TAIL)
        plsc.subcore_barrier()
        pltpu.sync_copy(acc_sh.at[pl.ds(base, RPT)],
                        out_hbm.at[cid, pl.ds(base, RPT)])

    return _deg_kernel, _agg_kernel


def _dinv_of(degp_blk):
    return lax.rsqrt(1.0 + jnp.sum(degp_blk, axis=1, keepdims=True))


def _stage_in_body(degp_ref, x_ref, w_ref, y_ref):
    dinv = _dinv_of(degp_ref[...])
    xw = jnp.dot(x_ref[...], w_ref[...], preferred_element_type=jnp.float32)
    y_ref[...] = xw * dinv


_stage_in = pl.pallas_call(
    _stage_in_body,
    grid=(GRID,),
    in_specs=[
        pl.BlockSpec((BLK, NW), lambda i: (i, 0)),
        pl.BlockSpec((BLK, D), lambda i: (i, 0)),
        pl.BlockSpec((D, D), lambda i: (0, 0)),
    ],
    out_specs=pl.BlockSpec((BLK, D), lambda i: (i, 0)),
    out_shape=jax.ShapeDtypeStruct((N, D), jnp.float32),
)


def _stage_mid_body(degp_ref, agg_ref, b_ref, w_ref, y_ref):
    dinv = _dinv_of(degp_ref[...])
    s = agg_ref[0] + agg_ref[1]
    h = jnp.maximum(s * dinv + b_ref[...], 0.0)
    y_ref[...] = (
        jnp.dot(h, w_ref[...], preferred_element_type=jnp.float32) * dinv
    )


_stage_mid = pl.pallas_call(
    _stage_mid_body,
    grid=(GRID,),
    in_specs=[
        pl.BlockSpec((BLK, NW), lambda i: (i, 0)),
        pl.BlockSpec((NC, BLK, D), lambda i: (0, i, 0)),
        pl.BlockSpec((1, D), lambda i: (0, 0)),
        pl.BlockSpec((D, D), lambda i: (0, 0)),
    ],
    out_specs=pl.BlockSpec((BLK, D), lambda i: (i, 0)),
    out_shape=jax.ShapeDtypeStruct((N, D), jnp.float32),
)


def _make_stage_out(dout):
    def _stage_out_body(degp_ref, agg_ref, b_ref, wl_ref, bl_ref,
                        nemb_ref, gsum_ref, gemb_ref):
        i = pl.program_id(0)
        dinv = _dinv_of(degp_ref[...])
        s = agg_ref[0] + agg_ref[1]
        nemb = s * dinv + b_ref[...]
        nemb_ref[...] = nemb

        @pl.when(i == 0)
        def _():
            gsum_ref[...] = jnp.zeros_like(gsum_ref)

        gsum_ref[...] += jnp.sum(nemb, axis=0, keepdims=True)

        @pl.when(i == GRID - 1)
        def _():
            g = gsum_ref[...] * (1.0 / N)
            gemb_ref[...] = (
                lax.dot_general(g, wl_ref[...], (((1,), (1,)), ((), ())),
                                preferred_element_type=jnp.float32)
                + bl_ref[...]
            )

    return pl.pallas_call(
        _stage_out_body,
        grid=(GRID,),
        in_specs=[
            pl.BlockSpec((BLK, NW), lambda i: (i, 0)),
            pl.BlockSpec((NC, BLK, D), lambda i: (0, i, 0)),
            pl.BlockSpec((1, D), lambda i: (0, 0)),
            pl.BlockSpec((dout, D), lambda i: (0, 0)),
            pl.BlockSpec((1, dout), lambda i: (0, 0)),
        ],
        out_specs=[
            pl.BlockSpec((BLK, D), lambda i: (i, 0)),
            pl.BlockSpec((1, D), lambda i: (0, 0)),
            pl.BlockSpec((1, dout), lambda i: (0, 0)),
        ],
        out_shape=[
            jax.ShapeDtypeStruct((N, D), jnp.float32),
            jax.ShapeDtypeStruct((1, D), jnp.float32),
            jax.ShapeDtypeStruct((1, dout), jnp.float32),
        ],
    )


def kernel(x, edge_index, batch, W1, b1, W2, b2, W3, b3, Wl, bl):
    dout = Wl.shape[0]
    src = edge_index[0].reshape(NW, NITER, K)
    dst = edge_index[1].reshape(NW, NITER, K)
    dst16 = edge_index[1].reshape(NW, EPT // 16, 16)
    zrows = jnp.zeros((RPT, D), jnp.float32)
    deg_kernel, agg_kernel = _sc_kernels()

    degp = jnp.transpose(deg_kernel(dst16))

    y1 = _stage_in(degp, x, W1)
    p1 = agg_kernel(y1, zrows, src, dst)
    y2 = _stage_mid(degp, p1, b1.reshape(1, D), W2)
    p2 = agg_kernel(y2, zrows, src, dst)
    y3 = _stage_mid(degp, p2, b2.reshape(1, D), W3)
    p3 = agg_kernel(y3, zrows, src, dst)
    nemb, _gsum, gemb = _make_stage_out(dout)(
        degp, p3, b3.reshape(1, D), Wl, bl.reshape(1, dout)
    )
    return (gemb, nemb)


# speedup vs baseline: 1.0717x; 1.0287x over previous
import functools

import jax
import jax.numpy as jnp
from jax import lax
from jax.experimental import pallas as pl
from jax.experimental.pallas import tpu as pltpu
from jax.experimental.pallas import tpu_sc as plsc

N = 10000
E = 320000
D = 128
NC = 2
NS = 16
NW = NC * NS
EPT = E // NW
K = 40
NITER = EPT // K
NPH = 2
NITH = NITER // NPH
GRP = 8
NGRP = NITH // GRP
TAIL = NITH - NGRP * GRP
RPT = N // NS
BLK = 2000
GRID = N // BLK

@functools.cache
def _sc_kernels():
    mesh = plsc.VectorSubcoreMesh(
        core_axis_name="c", subcore_axis_name="s",
        num_cores=NC, num_subcores=NS,
    )

    @functools.partial(
        pl.kernel,
        out_type=jax.ShapeDtypeStruct((NW, N), jnp.float32),
        mesh=mesh,
        compiler_params=pltpu.CompilerParams(needs_layout_passes=False),
        scratch_types=[
            pltpu.VMEM((EPT // 16, 16), jnp.int32),
            pltpu.VMEM((N,), jnp.float32),
        ],
    )
    def _deg_kernel(dst_hbm, out_hbm, idx_v, deg_v):
        cid = lax.axis_index("c")
        sid = lax.axis_index("s")
        wid = sid * NC + cid
        pltpu.sync_copy(dst_hbm.at[wid], idx_v)

        def zero_body(i, c):
            deg_v[pl.ds(i * 16, 16)] = jnp.zeros((16,), jnp.float32)
            return c

        lax.fori_loop(0, N // 16, zero_body, 0, unroll=4)

        ones = jnp.ones((16,), jnp.float32)

        def body(i, c):
            plsc.addupdate_scatter(deg_v, [idx_v[i]], ones)
            return c

        lax.fori_loop(0, EPT // 16, body, 0, unroll=4)
        pltpu.sync_copy(deg_v, out_hbm.at[wid])

    @functools.partial(
        pl.kernel,
        out_type=jax.ShapeDtypeStruct((NC, N, D), jnp.float32),
        mesh=mesh,
        compiler_params=pltpu.CompilerParams(
            needs_layout_passes=False, use_tc_tiling_on_sc=False),
        scratch_types=[
            pltpu.VMEM((NITH, K), jnp.int32),
            pltpu.VMEM((NITH, K), jnp.int32),
            pltpu.VMEM((GRP, K, D), jnp.float32),
            pltpu.VMEM_SHARED((N, D), jnp.float32),
            [pltpu.SemaphoreType.DMA] * GRP,
            [pltpu.SemaphoreType.DMA] * GRP,
            pltpu.SemaphoreType.DMA,
        ],
    )
    def _agg_kernel(y_hbm, z_hbm, src_hbm, dst_hbm, out_hbm,
                    si_v, di_v, rows_v, acc_sh, gsems, ssems, seed_sem):
        cid = lax.axis_index("c")
        sid = lax.axis_index("s")
        wid = sid * NC + cid
        base = sid * RPT

        @pl.when(cid == 0)
        def _():
            pltpu.async_copy(y_hbm.at[pl.ds(base, RPT)],
                             acc_sh.at[pl.ds(base, RPT)], seed_sem)

        @pl.when(cid != 0)
        def _():
            pltpu.async_copy(z_hbm, acc_sh.at[pl.ds(base, RPT)], seed_sem)

        pltpu.sync_copy(src_hbm.at[wid, pl.ds(0, NITH)], si_v)
        pltpu.sync_copy(dst_hbm.at[wid, pl.ds(0, NITH)], di_v)
        pltpu.make_async_copy(
            z_hbm, acc_sh.at[pl.ds(base, RPT)], seed_sem).wait()
        plsc.subcore_barrier()

        def group(j0, n):
            gh = [
                pltpu.async_copy(
                    y_hbm.at[si_v.at[j0 + b]], rows_v.at[b], gsems[b])
                for b in range(n)
            ]
            sh = []
            for b in range(n):
                gh[b].wait()
                sh.append(pltpu.async_copy(
                    rows_v.at[b], acc_sh.at[di_v.at[j0 + b]], ssems[b],
                    add=True))
            for h in sh:
                h.wait()

        def body(i, c):
            group(i * GRP, GRP)
            return c

        for ph in range(NPH):
            if ph:
                pltpu.sync_copy(src_hbm.at[wid, pl.ds(ph * NITH, NITH)], si_v)
                pltpu.sync_copy(dst_hbm.at[wid, pl.ds(ph * NITH, NITH)], di_v)
            lax.fori_loop(0, NGRP, body, 0)
            if TAIL:
                group(NGRP * GRP, TAIL)
        plsc.subcore_barrier()
        pltpu.sync_copy(acc_sh.at[pl.ds(base, RPT)],
                        out_hbm.at[cid, pl.ds(base, RPT)])

    return _deg_kernel, _agg_kernel


def _dinv_of(degp_blk):
    return lax.rsqrt(1.0 + jnp.sum(degp_blk, axis=1, keepdims=True))


def _stage_in_body(degp_ref, x_ref, w_ref, y_ref):
    dinv = _dinv_of(degp_ref[...])
    xw = jnp.dot(x_ref[...], w_ref[...], preferred_element_type=jnp.float32)
    y_ref[...] = xw * dinv


_stage_in = pl.pallas_call(
    _stage_in_body,
    grid=(GRID,),
    in_specs=[
        pl.BlockSpec((BLK, NW), lambda i: (i, 0)),
        pl.BlockSpec((BLK, D), lambda i: (i, 0)),
        pl.BlockSpec((D, D), lambda i: (0, 0)),
    ],
    out_specs=pl.BlockSpec((BLK, D), lambda i: (i, 0)),
    out_shape=jax.ShapeDtypeStruct((N, D), jnp.float32),
)


def _stage_mid_body(degp_ref, agg_ref, b_ref, w_ref, y_ref):
    dinv = _dinv_of(degp_ref[...])
    s = agg_ref[0] + agg_ref[1]
    h = jnp.maximum(s * dinv + b_ref[...], 0.0)
    y_ref[...] = (
        jnp.dot(h, w_ref[...], preferred_element_type=jnp.float32) * dinv
    )


_stage_mid = pl.pallas_call(
    _stage_mid_body,
    grid=(GRID,),
    in_specs=[
        pl.BlockSpec((BLK, NW), lambda i: (i, 0)),
        pl.BlockSpec((NC, BLK, D), lambda i: (0, i, 0)),
        pl.BlockSpec((1, D), lambda i: (0, 0)),
        pl.BlockSpec((D, D), lambda i: (0, 0)),
    ],
    out_specs=pl.BlockSpec((BLK, D), lambda i: (i, 0)),
    out_shape=jax.ShapeDtypeStruct((N, D), jnp.float32),
)


def _make_stage_out(dout):
    def _stage_out_body(degp_ref, agg_ref, b_ref, wl_ref, bl_ref,
                        nemb_ref, gsum_ref, gemb_ref):
        i = pl.program_id(0)
        dinv = _dinv_of(degp_ref[...])
        s = agg_ref[0] + agg_ref[1]
        nemb = s * dinv + b_ref[...]
        nemb_ref[...] = nemb

        @pl.when(i == 0)
        def _():
            gsum_ref[...] = jnp.zeros_like(gsum_ref)

        gsum_ref[...] += jnp.sum(nemb, axis=0, keepdims=True)

        @pl.when(i == GRID - 1)
        def _():
            g = gsum_ref[...] * (1.0 / N)
            gemb_ref[...] = (
                lax.dot_general(g, wl_ref[...], (((1,), (1,)), ((), ())),
                                preferred_element_type=jnp.float32)
                + bl_ref[...]
            )

    return pl.pallas_call(
        _stage_out_body,
        grid=(GRID,),
        in_specs=[
            pl.BlockSpec((BLK, NW), lambda i: (i, 0)),
            pl.BlockSpec((NC, BLK, D), lambda i: (0, i, 0)),
            pl.BlockSpec((1, D), lambda i: (0, 0)),
            pl.BlockSpec((dout, D), lambda i: (0, 0)),
            pl.BlockSpec((1, dout), lambda i: (0, 0)),
        ],
        out_specs=[
            pl.BlockSpec((BLK, D), lambda i: (i, 0)),
            pl.BlockSpec((1, D), lambda i: (0, 0)),
            pl.BlockSpec((1, dout), lambda i: (0, 0)),
        ],
        out_shape=[
            jax.ShapeDtypeStruct((N, D), jnp.float32),
            jax.ShapeDtypeStruct((1, D), jnp.float32),
            jax.ShapeDtypeStruct((1, dout), jnp.float32),
        ],
    )


def kernel(x, edge_index, batch, W1, b1, W2, b2, W3, b3, Wl, bl):
    dout = Wl.shape[0]
    src = edge_index[0].reshape(NW, NITER, K)
    dst = edge_index[1].reshape(NW, NITER, K)
    dst16 = edge_index[1].reshape(NW, EPT // 16, 16)
    zrows = jnp.zeros((RPT, D), jnp.float32)
    deg_kernel, agg_kernel = _sc_kernels()

    degp = jnp.transpose(deg_kernel(dst16))

    y1 = _stage_in(degp, x, W1)
    p1 = agg_kernel(y1, zrows, src, dst)
    y2 = _stage_mid(degp, p1, b1.reshape(1, D), W2)
    p2 = agg_kernel(y2, zrows, src, dst)
    y3 = _stage_mid(degp, p2, b2.reshape(1, D), W3)
    p3 = agg_kernel(y3, zrows, src, dst)
    nemb, _gsum, gemb = _make_stage_out(dout)(
        degp, p3, b3.reshape(1, D), Wl, bl.reshape(1, dout)
    )
    return (gemb, nemb)
